# baseline (speedup 1.0000x reference)
import jax
import jax.numpy as jnp
from jax.experimental import pallas as pl
from jax.experimental.pallas import tpu as pltpu


def _body(cid_ref, f_ref, c_ref, p_in, po_blk, co_ref):
    del p_in
    f = f_ref[...]
    nrm = jnp.sqrt(jnp.sum(f * f, axis=1, keepdims=True))
    po_blk[...] = (f / jnp.maximum(nrm, 1e-12))[None]
    co_ref[...] = c_ref[...]
    cid = cid_ref[0]
    co_ref[pl.ds(cid, 1), :] = jnp.ones((1, co_ref.shape[1]), jnp.int32)


def kernel(features, prototypes, counts, class_id):
    C, P, D = prototypes.shape
    n_add = min(features.shape[0], P)
    feats = features[:n_add]
    cid = jnp.asarray(class_id, jnp.int32).reshape((1,))

    grid_spec = pltpu.PrefetchScalarGridSpec(
        num_scalar_prefetch=1,
        grid=(1,),
        in_specs=[
            pl.BlockSpec((n_add, D), lambda i, c: (0, 0)),
            pl.BlockSpec((C, P), lambda i, c: (0, 0)),
            pl.BlockSpec(memory_space=pl.ANY),
        ],
        out_specs=[
            pl.BlockSpec((1, P, D), lambda i, c: (c[0], 0, 0)),
            pl.BlockSpec((C, P), lambda i, c: (0, 0)),
        ],
    )
    protos_out, counts_out = pl.pallas_call(
        _body,
        grid_spec=grid_spec,
        out_shape=[
            jax.ShapeDtypeStruct((C, P, D), jnp.float32),
            jax.ShapeDtypeStruct((C, P), jnp.int32),
        ],
        input_output_aliases={3: 0},
    )(cid, feats, counts, prototypes)
    return protos_out, counts_out


# speedup vs baseline: 1.0002x; 1.0002x over previous
import jax
import jax.numpy as jnp
from jax.experimental import pallas as pl
from jax.experimental.pallas import tpu as pltpu


def _body(cid_ref, f_ref, c_ref, p_in, po_blk, co_ref):
    del p_in
    f = f_ref[...]
    nrm = jnp.sqrt(jnp.sum(f * f, axis=1, keepdims=True))
    po_blk[...] = (f / jnp.maximum(nrm, 1e-12))[None]
    co_ref[...] = c_ref[...]
    cid = cid_ref[0]
    co_ref[pl.ds(cid, 1), :] = jnp.ones((1, co_ref.shape[1]), jnp.int32)


def kernel(features, prototypes, counts, class_id):
    C, P, D = prototypes.shape
    n_add = min(features.shape[0], P)
    feats = features[:n_add]
    cid = jnp.asarray(class_id, jnp.int32).reshape((1,))

    grid_spec = pltpu.PrefetchScalarGridSpec(
        num_scalar_prefetch=1,
        grid=(1,),
        in_specs=[
            pl.BlockSpec((n_add, D), lambda i, c: (0, 0)),
            pl.BlockSpec((C, P), lambda i, c: (0, 0)),
            pl.BlockSpec((1, P, D), lambda i, c: (0, 0, 0)),
        ],
        out_specs=[
            pl.BlockSpec((1, P, D), lambda i, c: (c[0], 0, 0)),
            pl.BlockSpec((C, P), lambda i, c: (0, 0)),
        ],
    )
    protos_out, counts_out = pl.pallas_call(
        _body,
        grid_spec=grid_spec,
        out_shape=[
            jax.ShapeDtypeStruct((C, P, D), jnp.float32),
            jax.ShapeDtypeStruct((C, P), jnp.int32),
        ],
        input_output_aliases={3: 0},
    )(cid, feats, counts, prototypes)
    return protos_out, counts_out
